# baseline (speedup 1.0000x reference)
import functools

import jax
import jax.numpy as jnp
from jax import lax
from jax.experimental import pallas as pl
from jax.experimental.pallas import tpu as pltpu
from jax.experimental.pallas import tpu_sc as plsc

N = 10000
NP = 10240
E = 320000
F = E + N
FP = 331776
D = 128
H = 4
C = 32
ED = 4

BN = 1024
BF = 2048
T1 = FP // 32
B1 = 1296
B2 = 2048
NEG = -1e9


def _colsum_body(ea_ref, o_ref):
    @pl.when(pl.program_id(0) == 0)
    def _():
        o_ref[...] = jnp.zeros_like(o_ref)
    o_ref[...] += jnp.sum(ea_ref[...], axis=0, keepdims=True)


def _colsum(ea2d):
    return pl.pallas_call(
        _colsum_body,
        grid=(10,),
        in_specs=[pl.BlockSpec((1000, 128), lambda i: (i, 0))],
        out_specs=pl.BlockSpec((1, 128), lambda i: (0, 0)),
        out_shape=jax.ShapeDtypeStruct((1, 128), jnp.float32),
    )(ea2d)


def _projlin_body(xT_ref, WpT_ref, bp_ref, WT_ref, A_ref,
                  hT_ref, xsT_ref, anT_ref):
    hv = (jnp.dot(WpT_ref[...], xT_ref[...],
                  preferred_element_type=jnp.float32) + bp_ref[...])
    hT_ref[...] = hv
    xs = jnp.dot(WT_ref[...], hv, preferred_element_type=jnp.float32)
    xsT_ref[...] = xs
    anT_ref[...] = jnp.dot(A_ref[...], xs, preferred_element_type=jnp.float32)


def _projlin(xT, WpT, bp_col, WT, A):
    return pl.pallas_call(
        _projlin_body,
        grid=(NP // BN,),
        in_specs=[
            pl.BlockSpec((3, BN), lambda i: (0, i)),
            pl.BlockSpec((D, 3), lambda i: (0, 0)),
            pl.BlockSpec((D, 1), lambda i: (0, 0)),
            pl.BlockSpec((D, D), lambda i: (0, 0)),
            pl.BlockSpec((2 * H, D), lambda i: (0, 0)),
        ],
        out_specs=[
            pl.BlockSpec((D, BN), lambda i: (0, i)),
            pl.BlockSpec((D, BN), lambda i: (0, i)),
            pl.BlockSpec((2 * H, BN), lambda i: (0, i)),
        ],
        out_shape=[
            jax.ShapeDtypeStruct((D, NP), jnp.float32),
            jax.ShapeDtypeStruct((D, NP), jnp.float32),
            jax.ShapeDtypeStruct((2 * H, NP), jnp.float32),
        ],
    )(xT, WpT, bp_col, WT, A)


def _post_math(accT, den, b, g, be, res):
    den128 = jnp.reshape(jnp.broadcast_to(den[:, None, :], (H, C, BN)),
                         (D, BN))
    o = accT / (den128 + 1e-16) + b
    mu = jnp.mean(o, axis=0, keepdims=True)
    var = jnp.mean((o - mu) ** 2, axis=0, keepdims=True)
    o = (o - mu) / jnp.sqrt(var + 1e-5) * g + be
    o = jnp.where(o > 0, o, jnp.exp(o) - 1.0)
    return o + res


def _postlin_body(accT_ref, den_ref, b_ref, g_ref, be_ref, res_ref,
                  WT_ref, A_ref, hT_ref, xsT_ref, anT_ref):
    hv = _post_math(accT_ref[...], den_ref[...], b_ref[...], g_ref[...],
                    be_ref[...], res_ref[...])
    hT_ref[...] = hv
    xs = jnp.dot(WT_ref[...], hv, preferred_element_type=jnp.float32)
    xsT_ref[...] = xs
    anT_ref[...] = jnp.dot(A_ref[...], xs, preferred_element_type=jnp.float32)


def _postlin(accT, den, b_col, g_col, be_col, resT, WT, A):
    return pl.pallas_call(
        _postlin_body,
        grid=(NP // BN,),
        in_specs=[
            pl.BlockSpec((D, BN), lambda i: (0, i)),
            pl.BlockSpec((H, BN), lambda i: (0, i)),
            pl.BlockSpec((D, 1), lambda i: (0, 0)),
            pl.BlockSpec((D, 1), lambda i: (0, 0)),
            pl.BlockSpec((D, 1), lambda i: (0, 0)),
            pl.BlockSpec((D, BN), lambda i: (0, i)),
            pl.BlockSpec((D, D), lambda i: (0, 0)),
            pl.BlockSpec((2 * H, D), lambda i: (0, 0)),
        ],
        out_specs=[
            pl.BlockSpec((D, BN), lambda i: (0, i)),
            pl.BlockSpec((D, BN), lambda i: (0, i)),
            pl.BlockSpec((2 * H, BN), lambda i: (0, i)),
        ],
        out_shape=[
            jax.ShapeDtypeStruct((D, NP), jnp.float32),
            jax.ShapeDtypeStruct((D, NP), jnp.float32),
            jax.ShapeDtypeStruct((2 * H, NP), jnp.float32),
        ],
    )(accT, den, b_col, g_col, be_col, resT, WT, A)


def _post_body(accT_ref, den_ref, b_ref, g_ref, be_ref, res_ref, o_ref):
    o_ref[...] = _post_math(accT_ref[...], den_ref[...], b_ref[...],
                            g_ref[...], be_ref[...], res_ref[...])


def _post(accT, den, b_col, g_col, be_col, resT):
    return pl.pallas_call(
        _post_body,
        grid=(NP // BN,),
        in_specs=[
            pl.BlockSpec((D, BN), lambda i: (0, i)),
            pl.BlockSpec((H, BN), lambda i: (0, i)),
            pl.BlockSpec((D, 1), lambda i: (0, 0)),
            pl.BlockSpec((D, 1), lambda i: (0, 0)),
            pl.BlockSpec((D, 1), lambda i: (0, 0)),
            pl.BlockSpec((D, BN), lambda i: (0, i)),
        ],
        out_specs=pl.BlockSpec((D, BN), lambda i: (0, i)),
        out_shape=jax.ShapeDtypeStruct((D, NP), jnp.float32),
    )(accT, den, b_col, g_col, be_col, resT)


def _rep16(x, pat):
    return lax.gather(
        x,
        pat.reshape(16, 1),
        lax.GatherDimensionNumbers(
            offset_dims=(), collapsed_slice_dims=(0,), start_index_map=(0,)),
        slice_sizes=(1,),
        mode=lax.GatherScatterMode.PROMISE_IN_BOUNDS,
    )


def _sc_logits(anT_flat, eaFT_flat, veT_flat, srcF, dstF):
    mesh = plsc.VectorSubcoreMesh(core_axis_name="c", subcore_axis_name="s")

    @functools.partial(
        pl.kernel,
        out_type=jax.ShapeDtypeStruct((H * FP,), jnp.float32),
        mesh=mesh,
        compiler_params=pltpu.CompilerParams(needs_layout_passes=False),
        scratch_types=[
            pltpu.VMEM((2 * H * NP,), jnp.float32),
            pltpu.VMEM((H * ED * 16,), jnp.float32),
            pltpu.VMEM((B1,), jnp.int32),
            pltpu.VMEM((B1,), jnp.int32),
            pltpu.VMEM((ED * B1,), jnp.float32),
            pltpu.VMEM((H * B1,), jnp.float32),
        ],
    )
    def k(anT_hbm, eaFT_hbm, veT_hbm, src_hbm, dst_hbm, exT_hbm,
          an_v, ve_v, src_v, dst_v, ea_v, ex_v):
        cid = lax.axis_index("c")
        sid = lax.axis_index("s")
        wid = sid * 2 + cid
        pltpu.sync_copy(anT_hbm, an_v)
        pltpu.sync_copy(veT_hbm, ve_v)
        ve = [[ve_v[pl.ds((h * ED + d) * 16, 16)]
               for d in range(ED)] for h in range(H)]
        lane = lax.iota(jnp.int32, 16)

        def chunk(ci, carry):
            base = wid * T1 + ci * B1
            pltpu.sync_copy(src_hbm.at[pl.ds(base, B1)], src_v)
            pltpu.sync_copy(dst_hbm.at[pl.ds(base, B1)], dst_v)
            for dd in range(ED):
                pltpu.sync_copy(eaFT_hbm.at[pl.ds(base + dd * FP, B1)],
                                ea_v.at[pl.ds(dd * B1, B1)])

            def step(j):
                off = j * 16
                s16 = src_v[pl.ds(off, 16)]
                d16 = dst_v[pl.ds(off, 16)]
                ea = [ea_v[pl.ds(d * B1 + off, 16)] for d in range(ED)]
                inb = (base + off + lane) < F
                for h in range(H):
                    asv = plsc.load_gather(an_v, [s16 + h * NP])
                    adv = plsc.load_gather(an_v, [d16 + (H + h) * NP])
                    ae = (ve[h][0] * ea[0] + ve[h][1] * ea[1]
                          + ve[h][2] * ea[2] + ve[h][3] * ea[3])
                    al = asv + adv + ae
                    al = jnp.where(al > 0, al, 0.2 * al)
                    ex_v[pl.ds(h * B1 + off, 16)] = jnp.where(
                        inb, jnp.exp(al), 0.0)

            plsc.parallel_loop(0, B1 // 16, unroll=4)(step)
            for hh in range(H):
                pltpu.sync_copy(ex_v.at[pl.ds(hh * B1, B1)],
                                exT_hbm.at[pl.ds(base + hh * FP, B1)])
            return carry

        lax.fori_loop(0, T1 // B1, chunk, 0)

    return k(anT_flat, eaFT_flat, veT_flat, srcF, dstF)


def _sc_aggregate(xsT_flat, exT_flat, srcF, dstF):
    mesh = plsc.VectorSubcoreMesh(core_axis_name="c", subcore_axis_name="s")

    @functools.partial(
        pl.kernel,
        out_type=(
            jax.ShapeDtypeStruct((D * NP,), jnp.float32),
            jax.ShapeDtypeStruct((H * NP,), jnp.float32),
        ),
        mesh=mesh,
        compiler_params=pltpu.CompilerParams(needs_layout_passes=False),
        scratch_types=[
            pltpu.VMEM((4 * NP,), jnp.float32),
            pltpu.VMEM((4 * NP,), jnp.float32),
            pltpu.VMEM((NP,), jnp.float32),
            pltpu.VMEM((2 * B2,), jnp.int32),
            pltpu.VMEM((2 * B2,), jnp.int32),
            pltpu.VMEM((2 * B2,), jnp.float32),
            pltpu.SemaphoreType.DMA,
            pltpu.SemaphoreType.DMA,
        ],
    )
    def k(xsT_hbm, exT_hbm, src_hbm, dst_hbm, accT_hbm, den_hbm,
          xs_v, acc_v, den_v, src_v, dst_v, ex_v, sem0, sem1):
        cid = lax.axis_index("c")
        sid = lax.axis_index("s")
        wid = sid * 2 + cid
        h = wid // 8
        g = wid % 8

        pltpu.sync_copy(xsT_hbm.at[pl.ds(wid * (4 * NP), 4 * NP)], xs_v)

        zz = jnp.zeros((16,), jnp.float32)

        def za(i, carry):
            acc_v[pl.ds(i * 16, 16)] = zz
            return carry
        lax.fori_loop(0, 4 * NP // 16, za, 0)

        def zd(i, carry):
            den_v[pl.ds(i * 16, 16)] = zz
            return carry
        lax.fori_loop(0, NP // 16, zd, 0)

        NC2 = FP // B2
        sems = (sem0, sem1)

        def issue(ci, slot):
            base = ci * B2
            boff = slot * B2
            sem = sems[slot]
            pltpu.async_copy(src_hbm.at[pl.ds(base, B2)],
                             src_v.at[pl.ds(boff, B2)], sem)
            pltpu.async_copy(dst_hbm.at[pl.ds(base, B2)],
                             dst_v.at[pl.ds(boff, B2)], sem)
            pltpu.async_copy(exT_hbm.at[pl.ds(h * FP + base, B2)],
                             ex_v.at[pl.ds(boff, B2)], sem)

        def wait_slot(slot):
            boff = slot * B2
            sem = sems[slot]
            pltpu.make_async_copy(src_hbm.at[pl.ds(0, B2)],
                                  src_v.at[pl.ds(boff, B2)], sem).wait()
            pltpu.make_async_copy(dst_hbm.at[pl.ds(0, B2)],
                                  dst_v.at[pl.ds(boff, B2)], sem).wait()
            pltpu.make_async_copy(exT_hbm.at[pl.ds(0, B2)],
                                  ex_v.at[pl.ds(boff, B2)], sem).wait()

        issue(0, 0)
        issue(1, 1)

        def pair(pi, carry):
            for slot in range(2):
                ci = pi * 2 + slot
                boff = slot * B2
                wait_slot(slot)

                def step(j):
                    off = boff + j * 16
                    s16 = src_v[pl.ds(off, 16)]
                    d16 = dst_v[pl.ds(off, 16)]
                    e16 = ex_v[pl.ds(off, 16)]
                    plsc.addupdate_scatter(den_v, [d16], e16)
                    for c in range(4):
                        gv = plsc.load_gather(xs_v, [s16 + c * NP])
                        plsc.addupdate_scatter(
                            acc_v, [d16 + c * NP], gv * e16)

                plsc.parallel_loop(0, B2 // 16, unroll=16)(step)

                @pl.when(ci + 2 < NC2)
                def _():
                    issue(ci + 2, slot)
            return carry

        lax.fori_loop(0, NC2 // 2, pair, 0)

        pltpu.sync_copy(acc_v, accT_hbm.at[pl.ds(wid * (4 * NP), 4 * NP)])

        @pl.when(g == 0)
        def _():
            pltpu.sync_copy(den_v, den_hbm.at[pl.ds(h * NP, NP)])

    return k(xsT_flat, exT_flat, srcF, dstF)


def kernel(x, edge_index, edge_attr, Wp, bp,
           W0, as0, ad0, ae0, We0, b0, g0, be0,
           W1, as1, ad1, ae1, We1, b1, g1, be1):
    src, dst = edge_index[0], edge_index[1]
    loop = jnp.arange(N, dtype=src.dtype)
    padi = jnp.zeros((FP - F,), src.dtype)
    srcF = jnp.concatenate([src, loop, padi])
    dstF = jnp.concatenate([dst, loop, padi])

    colsum = _colsum(edge_attr.reshape(E * ED // 128, 128))
    m4 = colsum.reshape(C, ED).sum(axis=0) / E
    eaFT = jnp.concatenate(
        [edge_attr.T,
         jnp.broadcast_to(m4[:, None], (ED, N)),
         jnp.zeros((ED, FP - F), jnp.float32)], axis=1).reshape(-1)

    xT = jnp.pad(x.T, ((0, 0), (0, NP - N)))

    eye = jnp.eye(H, dtype=jnp.float32)

    def mkA(a_s, a_d):
        return jnp.concatenate([
            (eye[:, :, None] * a_s[None, :, :]).reshape(H, D),
            (eye[:, :, None] * a_d[None, :, :]).reshape(H, D)], axis=0)

    def mkVeT(We, a_e):
        veT = jnp.einsum("dhc,hc->dh", We.reshape(ED, H, C), a_e).T.reshape(-1)
        return jnp.broadcast_to(veT[:, None], (H * ED, 16)).reshape(-1)

    hT, xsT, anT = _projlin(xT, Wp.T, bp[:, None], W0.T, mkA(as0, ad0))
    exT = _sc_logits(anT.reshape(-1), eaFT, mkVeT(We0, ae0), srcF, dstF)
    accT, den = _sc_aggregate(xsT.reshape(-1), exT, srcF, dstF)

    hT, xsT, anT = _postlin(accT.reshape(D, NP), den.reshape(H, NP),
                            b0[:, None], g0[:, None], be0[:, None], hT,
                            W1.T, mkA(as1, ad1))
    exT = _sc_logits(anT.reshape(-1), eaFT, mkVeT(We1, ae1), srcF, dstF)
    accT, den = _sc_aggregate(xsT.reshape(-1), exT, srcF, dstF)

    hT = _post(accT.reshape(D, NP), den.reshape(H, NP),
               b1[:, None], g1[:, None], be1[:, None], hT)

    return hT[:, :N].T


# speedup vs baseline: 1.0587x; 1.0587x over previous
import functools

import jax
import jax.numpy as jnp
from jax import lax
from jax.experimental import pallas as pl
from jax.experimental.pallas import tpu as pltpu
from jax.experimental.pallas import tpu_sc as plsc

N = 10000
NP = 10240
E = 320000
F = E + N
FP = 331776
D = 128
H = 4
C = 32
ED = 4

BN = 1024
BF = 2048
T1 = FP // 32
B1 = 1296
B2 = 2048
NEG = -1e9


def _colsum_body(ea_ref, o_ref):
    @pl.when(pl.program_id(0) == 0)
    def _():
        o_ref[...] = jnp.zeros_like(o_ref)
    o_ref[...] += jnp.sum(ea_ref[...], axis=0, keepdims=True)


def _colsum(ea2d):
    return pl.pallas_call(
        _colsum_body,
        grid=(10,),
        in_specs=[pl.BlockSpec((1000, 128), lambda i: (i, 0))],
        out_specs=pl.BlockSpec((1, 128), lambda i: (0, 0)),
        out_shape=jax.ShapeDtypeStruct((1, 128), jnp.float32),
    )(ea2d)


def _projlin_body(xT_ref, WpT_ref, bp_ref, WT_ref, A_ref,
                  hT_ref, xsT_ref, anT_ref):
    hv = (jnp.dot(WpT_ref[...], xT_ref[...],
                  preferred_element_type=jnp.float32) + bp_ref[...])
    hT_ref[...] = hv
    xs = jnp.dot(WT_ref[...], hv, preferred_element_type=jnp.float32)
    xsT_ref[...] = xs
    anT_ref[...] = jnp.dot(A_ref[...], xs, preferred_element_type=jnp.float32)


def _projlin(xT, WpT, bp_col, WT, A):
    return pl.pallas_call(
        _projlin_body,
        grid=(NP // BN,),
        in_specs=[
            pl.BlockSpec((3, BN), lambda i: (0, i)),
            pl.BlockSpec((D, 3), lambda i: (0, 0)),
            pl.BlockSpec((D, 1), lambda i: (0, 0)),
            pl.BlockSpec((D, D), lambda i: (0, 0)),
            pl.BlockSpec((2 * H, D), lambda i: (0, 0)),
        ],
        out_specs=[
            pl.BlockSpec((D, BN), lambda i: (0, i)),
            pl.BlockSpec((D, BN), lambda i: (0, i)),
            pl.BlockSpec((2 * H, BN), lambda i: (0, i)),
        ],
        out_shape=[
            jax.ShapeDtypeStruct((D, NP), jnp.float32),
            jax.ShapeDtypeStruct((D, NP), jnp.float32),
            jax.ShapeDtypeStruct((2 * H, NP), jnp.float32),
        ],
    )(xT, WpT, bp_col, WT, A)


def _post_math(accT, den, b, g, be, res):
    den128 = jnp.reshape(jnp.broadcast_to(den[:, None, :], (H, C, BN)),
                         (D, BN))
    o = accT / (den128 + 1e-16) + b
    mu = jnp.mean(o, axis=0, keepdims=True)
    var = jnp.mean((o - mu) ** 2, axis=0, keepdims=True)
    o = (o - mu) / jnp.sqrt(var + 1e-5) * g + be
    o = jnp.where(o > 0, o, jnp.exp(o) - 1.0)
    return o + res


def _postlin_body(accT_ref, den_ref, b_ref, g_ref, be_ref, res_ref,
                  WT_ref, A_ref, hT_ref, xsT_ref, anT_ref):
    hv = _post_math(accT_ref[...], den_ref[...], b_ref[...], g_ref[...],
                    be_ref[...], res_ref[...])
    hT_ref[...] = hv
    xs = jnp.dot(WT_ref[...], hv, preferred_element_type=jnp.float32)
    xsT_ref[...] = xs
    anT_ref[...] = jnp.dot(A_ref[...], xs, preferred_element_type=jnp.float32)


def _postlin(accT, den, b_col, g_col, be_col, resT, WT, A):
    return pl.pallas_call(
        _postlin_body,
        grid=(NP // BN,),
        in_specs=[
            pl.BlockSpec((D, BN), lambda i: (0, i)),
            pl.BlockSpec((H, BN), lambda i: (0, i)),
            pl.BlockSpec((D, 1), lambda i: (0, 0)),
            pl.BlockSpec((D, 1), lambda i: (0, 0)),
            pl.BlockSpec((D, 1), lambda i: (0, 0)),
            pl.BlockSpec((D, BN), lambda i: (0, i)),
            pl.BlockSpec((D, D), lambda i: (0, 0)),
            pl.BlockSpec((2 * H, D), lambda i: (0, 0)),
        ],
        out_specs=[
            pl.BlockSpec((D, BN), lambda i: (0, i)),
            pl.BlockSpec((D, BN), lambda i: (0, i)),
            pl.BlockSpec((2 * H, BN), lambda i: (0, i)),
        ],
        out_shape=[
            jax.ShapeDtypeStruct((D, NP), jnp.float32),
            jax.ShapeDtypeStruct((D, NP), jnp.float32),
            jax.ShapeDtypeStruct((2 * H, NP), jnp.float32),
        ],
    )(accT, den, b_col, g_col, be_col, resT, WT, A)


def _post_body(accT_ref, den_ref, b_ref, g_ref, be_ref, res_ref, o_ref):
    o_ref[...] = _post_math(accT_ref[...], den_ref[...], b_ref[...],
                            g_ref[...], be_ref[...], res_ref[...])


def _post(accT, den, b_col, g_col, be_col, resT):
    return pl.pallas_call(
        _post_body,
        grid=(NP // BN,),
        in_specs=[
            pl.BlockSpec((D, BN), lambda i: (0, i)),
            pl.BlockSpec((H, BN), lambda i: (0, i)),
            pl.BlockSpec((D, 1), lambda i: (0, 0)),
            pl.BlockSpec((D, 1), lambda i: (0, 0)),
            pl.BlockSpec((D, 1), lambda i: (0, 0)),
            pl.BlockSpec((D, BN), lambda i: (0, i)),
        ],
        out_specs=pl.BlockSpec((D, BN), lambda i: (0, i)),
        out_shape=jax.ShapeDtypeStruct((D, NP), jnp.float32),
    )(accT, den, b_col, g_col, be_col, resT)


def _rep16(x, pat):
    return lax.gather(
        x,
        pat.reshape(16, 1),
        lax.GatherDimensionNumbers(
            offset_dims=(), collapsed_slice_dims=(0,), start_index_map=(0,)),
        slice_sizes=(1,),
        mode=lax.GatherScatterMode.PROMISE_IN_BOUNDS,
    )


def _sc_logits(anT_flat, eaFT_flat, veT_flat, srcF, dstF):
    mesh = plsc.VectorSubcoreMesh(core_axis_name="c", subcore_axis_name="s")

    @functools.partial(
        pl.kernel,
        out_type=jax.ShapeDtypeStruct((H * FP,), jnp.float32),
        mesh=mesh,
        compiler_params=pltpu.CompilerParams(needs_layout_passes=False),
        scratch_types=[
            pltpu.VMEM((2 * H * NP,), jnp.float32),
            pltpu.VMEM((H * ED * 16,), jnp.float32),
            pltpu.VMEM((2 * B1,), jnp.int32),
            pltpu.VMEM((2 * B1,), jnp.int32),
            pltpu.VMEM((2 * ED * B1,), jnp.float32),
            pltpu.VMEM((2 * H * B1,), jnp.float32),
            pltpu.SemaphoreType.DMA,
            pltpu.SemaphoreType.DMA,
            pltpu.SemaphoreType.DMA,
            pltpu.SemaphoreType.DMA,
        ],
    )
    def k(anT_hbm, eaFT_hbm, veT_hbm, src_hbm, dst_hbm, exT_hbm,
          an_v, ve_v, src_v, dst_v, ea_v, ex_v, si0, si1, so0, so1):
        cid = lax.axis_index("c")
        sid = lax.axis_index("s")
        wid = sid * 2 + cid
        pltpu.sync_copy(anT_hbm, an_v)
        pltpu.sync_copy(veT_hbm, ve_v)
        ve = [[ve_v[pl.ds((h * ED + d) * 16, 16)]
               for d in range(ED)] for h in range(H)]
        lane = lax.iota(jnp.int32, 16)

        NC1 = T1 // B1
        sin = (si0, si1)
        sout = (so0, so1)

        def issue_in(ci, slot):
            base = wid * T1 + ci * B1
            sem = sin[slot]
            pltpu.async_copy(src_hbm.at[pl.ds(base, B1)],
                             src_v.at[pl.ds(slot * B1, B1)], sem)
            pltpu.async_copy(dst_hbm.at[pl.ds(base, B1)],
                             dst_v.at[pl.ds(slot * B1, B1)], sem)
            for dd in range(ED):
                pltpu.async_copy(
                    eaFT_hbm.at[pl.ds(base + dd * FP, B1)],
                    ea_v.at[pl.ds(slot * ED * B1 + dd * B1, B1)], sem)

        def wait_in(slot):
            sem = sin[slot]
            pltpu.make_async_copy(src_hbm.at[pl.ds(0, B1)],
                                  src_v.at[pl.ds(slot * B1, B1)], sem).wait()
            pltpu.make_async_copy(dst_hbm.at[pl.ds(0, B1)],
                                  dst_v.at[pl.ds(slot * B1, B1)], sem).wait()
            for dd in range(ED):
                pltpu.make_async_copy(
                    eaFT_hbm.at[pl.ds(0, B1)],
                    ea_v.at[pl.ds(slot * ED * B1 + dd * B1, B1)], sem).wait()

        def issue_out(ci, slot):
            base = wid * T1 + ci * B1
            sem = sout[slot]
            for hh in range(H):
                pltpu.async_copy(
                    ex_v.at[pl.ds(slot * H * B1 + hh * B1, B1)],
                    exT_hbm.at[pl.ds(base + hh * FP, B1)], sem)

        def wait_out(slot):
            sem = sout[slot]
            for hh in range(H):
                pltpu.make_async_copy(
                    exT_hbm.at[pl.ds(0, B1)],
                    ex_v.at[pl.ds(slot * H * B1 + hh * B1, B1)], sem).wait()

        issue_in(0, 0)
        issue_in(1, 1)

        def pair(pi, carry):
            for slot in range(2):
                ci = pi * 2 + slot
                base = wid * T1 + ci * B1
                boff = slot * B1
                eoff = slot * ED * B1
                xoff = slot * H * B1
                wait_in(slot)

                @pl.when(ci >= 2)
                def _():
                    wait_out(slot)

                def step(j):
                    off = j * 16
                    s16 = src_v[pl.ds(boff + off, 16)]
                    d16 = dst_v[pl.ds(boff + off, 16)]
                    ea = [ea_v[pl.ds(eoff + d * B1 + off, 16)]
                          for d in range(ED)]
                    inb = (base + off + lane) < F
                    for h in range(H):
                        asv = plsc.load_gather(an_v, [s16 + h * NP])
                        adv = plsc.load_gather(an_v, [d16 + (H + h) * NP])
                        ae = (ve[h][0] * ea[0] + ve[h][1] * ea[1]
                              + ve[h][2] * ea[2] + ve[h][3] * ea[3])
                        al = asv + adv + ae
                        al = jnp.where(al > 0, al, 0.2 * al)
                        ex_v[pl.ds(xoff + h * B1 + off, 16)] = jnp.where(
                            inb, jnp.exp(al), 0.0)

                plsc.parallel_loop(0, B1 // 16, unroll=4)(step)
                issue_out(ci, slot)

                @pl.when(ci + 2 < NC1)
                def _():
                    issue_in(ci + 2, slot)
            return carry

        lax.fori_loop(0, NC1 // 2, pair, 0)
        wait_out(0)
        wait_out(1)

    return k(anT_flat, eaFT_flat, veT_flat, srcF, dstF)


def _sc_aggregate(xsT_flat, exT_flat, srcF, dstF):
    mesh = plsc.VectorSubcoreMesh(core_axis_name="c", subcore_axis_name="s")

    @functools.partial(
        pl.kernel,
        out_type=(
            jax.ShapeDtypeStruct((D * NP,), jnp.float32),
            jax.ShapeDtypeStruct((H * NP,), jnp.float32),
        ),
        mesh=mesh,
        compiler_params=pltpu.CompilerParams(needs_layout_passes=False),
        scratch_types=[
            pltpu.VMEM((4 * NP,), jnp.float32),
            pltpu.VMEM((4 * NP,), jnp.float32),
            pltpu.VMEM((NP,), jnp.float32),
            pltpu.VMEM((2 * B2,), jnp.int32),
            pltpu.VMEM((2 * B2,), jnp.int32),
            pltpu.VMEM((2 * B2,), jnp.float32),
            pltpu.SemaphoreType.DMA,
            pltpu.SemaphoreType.DMA,
        ],
    )
    def k(xsT_hbm, exT_hbm, src_hbm, dst_hbm, accT_hbm, den_hbm,
          xs_v, acc_v, den_v, src_v, dst_v, ex_v, sem0, sem1):
        cid = lax.axis_index("c")
        sid = lax.axis_index("s")
        wid = sid * 2 + cid
        h = wid // 8
        g = wid % 8

        pltpu.sync_copy(xsT_hbm.at[pl.ds(wid * (4 * NP), 4 * NP)], xs_v)

        zz = jnp.zeros((16,), jnp.float32)

        def za(i, carry):
            acc_v[pl.ds(i * 16, 16)] = zz
            return carry
        lax.fori_loop(0, 4 * NP // 16, za, 0)

        def zd(i, carry):
            den_v[pl.ds(i * 16, 16)] = zz
            return carry
        lax.fori_loop(0, NP // 16, zd, 0)

        NC2 = FP // B2
        sems = (sem0, sem1)

        def issue(ci, slot):
            base = ci * B2
            boff = slot * B2
            sem = sems[slot]
            pltpu.async_copy(src_hbm.at[pl.ds(base, B2)],
                             src_v.at[pl.ds(boff, B2)], sem)
            pltpu.async_copy(dst_hbm.at[pl.ds(base, B2)],
                             dst_v.at[pl.ds(boff, B2)], sem)
            pltpu.async_copy(exT_hbm.at[pl.ds(h * FP + base, B2)],
                             ex_v.at[pl.ds(boff, B2)], sem)

        def wait_slot(slot):
            boff = slot * B2
            sem = sems[slot]
            pltpu.make_async_copy(src_hbm.at[pl.ds(0, B2)],
                                  src_v.at[pl.ds(boff, B2)], sem).wait()
            pltpu.make_async_copy(dst_hbm.at[pl.ds(0, B2)],
                                  dst_v.at[pl.ds(boff, B2)], sem).wait()
            pltpu.make_async_copy(exT_hbm.at[pl.ds(0, B2)],
                                  ex_v.at[pl.ds(boff, B2)], sem).wait()

        issue(0, 0)
        issue(1, 1)

        def pair(pi, carry):
            for slot in range(2):
                ci = pi * 2 + slot
                boff = slot * B2
                wait_slot(slot)

                def step(j):
                    off = boff + j * 16
                    s16 = src_v[pl.ds(off, 16)]
                    d16 = dst_v[pl.ds(off, 16)]
                    e16 = ex_v[pl.ds(off, 16)]
                    plsc.addupdate_scatter(den_v, [d16], e16)
                    for c in range(4):
                        gv = plsc.load_gather(xs_v, [s16 + c * NP])
                        plsc.addupdate_scatter(
                            acc_v, [d16 + c * NP], gv * e16)

                plsc.parallel_loop(0, B2 // 16, unroll=16)(step)

                @pl.when(ci + 2 < NC2)
                def _():
                    issue(ci + 2, slot)
            return carry

        lax.fori_loop(0, NC2 // 2, pair, 0)

        pltpu.sync_copy(acc_v, accT_hbm.at[pl.ds(wid * (4 * NP), 4 * NP)])

        @pl.when(g == 0)
        def _():
            pltpu.sync_copy(den_v, den_hbm.at[pl.ds(h * NP, NP)])

    return k(xsT_flat, exT_flat, srcF, dstF)


def kernel(x, edge_index, edge_attr, Wp, bp,
           W0, as0, ad0, ae0, We0, b0, g0, be0,
           W1, as1, ad1, ae1, We1, b1, g1, be1):
    src, dst = edge_index[0], edge_index[1]
    loop = jnp.arange(N, dtype=src.dtype)
    padi = jnp.zeros((FP - F,), src.dtype)
    srcF = jnp.concatenate([src, loop, padi])
    dstF = jnp.concatenate([dst, loop, padi])

    colsum = _colsum(edge_attr.reshape(E * ED // 128, 128))
    m4 = colsum.reshape(C, ED).sum(axis=0) / E
    eaFT = jnp.concatenate(
        [edge_attr.T,
         jnp.broadcast_to(m4[:, None], (ED, N)),
         jnp.zeros((ED, FP - F), jnp.float32)], axis=1).reshape(-1)

    xT = jnp.pad(x.T, ((0, 0), (0, NP - N)))

    eye = jnp.eye(H, dtype=jnp.float32)

    def mkA(a_s, a_d):
        return jnp.concatenate([
            (eye[:, :, None] * a_s[None, :, :]).reshape(H, D),
            (eye[:, :, None] * a_d[None, :, :]).reshape(H, D)], axis=0)

    def mkVeT(We, a_e):
        veT = jnp.einsum("dhc,hc->dh", We.reshape(ED, H, C), a_e).T.reshape(-1)
        return jnp.broadcast_to(veT[:, None], (H * ED, 16)).reshape(-1)

    hT, xsT, anT = _projlin(xT, Wp.T, bp[:, None], W0.T, mkA(as0, ad0))
    exT = _sc_logits(anT.reshape(-1), eaFT, mkVeT(We0, ae0), srcF, dstF)
    accT, den = _sc_aggregate(xsT.reshape(-1), exT, srcF, dstF)

    hT, xsT, anT = _postlin(accT.reshape(D, NP), den.reshape(H, NP),
                            b0[:, None], g0[:, None], be0[:, None], hT,
                            W1.T, mkA(as1, ad1))
    exT = _sc_logits(anT.reshape(-1), eaFT, mkVeT(We1, ae1), srcF, dstF)
    accT, den = _sc_aggregate(xsT.reshape(-1), exT, srcF, dstF)

    hT = _post(accT.reshape(D, NP), den.reshape(H, NP),
               b1[:, None], g1[:, None], be1[:, None], hT)

    return hT[:, :N].T
